# Initial kernel scaffold; baseline (speedup 1.0000x reference)
import jax
import jax.numpy as jnp
from jax.experimental import pallas as pl


def kernel(qoi_features, W1, b1, W2, b2, W3, b3):
    raise NotImplementedError("write your pallas kernel here")


# speedup vs baseline: 6.6232x; 6.6232x over previous
import functools

import jax
import jax.numpy as jnp
from jax.experimental import pallas as pl

_BATCH = 16384
_IN = 256
_HID = 256
_NSLOT = 24
_K = 6
_TILE = 1024

_NEG = -3e38


def _body(x_ref, w1_ref, b1_ref, w2_ref, b2_ref, w3_ref, b3_ref, o_ref):
    x = x_ref[...]
    h = jnp.dot(x, w1_ref[...], preferred_element_type=jnp.float32) + b1_ref[...]
    h = jnp.maximum(h, 0.0)
    h = jnp.dot(h, w2_ref[...], preferred_element_type=jnp.float32) + b2_ref[...]
    h = jnp.maximum(h, 0.0)
    logits = jnp.dot(h, w3_ref[...], preferred_element_type=jnp.float32) + b3_ref[...]

    col = jax.lax.broadcasted_iota(jnp.int32, logits.shape, 1)
    selectable = (col != 0) & (col != 12)
    work = jnp.where(selectable, logits, _NEG)
    acc = jnp.where(selectable, 0.0, 1.0)
    for _ in range(_K):
        m = jnp.max(work, axis=1, keepdims=True)
        first = jnp.min(jnp.where(work == m, col, _NSLOT), axis=1, keepdims=True)
        pick = col == first
        acc = jnp.where(pick, 1.0, acc)
        work = jnp.where(pick, _NEG, work)
    o_ref[...] = acc


@functools.partial(jax.jit, static_argnames=("interpret",))
def kernel(qoi_features, W1, b1, W2, b2, W3, b3, interpret=False):
    out = pl.pallas_call(
        _body,
        grid=(_BATCH // _TILE,),
        in_specs=[
            pl.BlockSpec((_TILE, _IN), lambda i: (i, 0)),
            pl.BlockSpec((_IN, _HID), lambda i: (0, 0)),
            pl.BlockSpec((1, _HID), lambda i: (0, 0)),
            pl.BlockSpec((_HID, _HID), lambda i: (0, 0)),
            pl.BlockSpec((1, _HID), lambda i: (0, 0)),
            pl.BlockSpec((_HID, _NSLOT), lambda i: (0, 0)),
            pl.BlockSpec((1, _NSLOT), lambda i: (0, 0)),
        ],
        out_specs=pl.BlockSpec((_TILE, _NSLOT), lambda i: (i, 0)),
        out_shape=jax.ShapeDtypeStruct((_BATCH, _NSLOT), jnp.float32),
        interpret=interpret,
    )(qoi_features, W1, b1.reshape(1, _HID), W2, b2.reshape(1, _HID),
      W3, b3.reshape(1, _NSLOT))
    return out.reshape(_BATCH, 2, 12)
